# baseline (speedup 1.0000x reference)
import jax
import jax.numpy as jnp
from jax import lax
from jax.experimental import pallas as pl
from jax.experimental.pallas import tpu as pltpu
from jax.experimental.pallas import tpu_sc as plsc

_NC = 2
_NS = 16
_NW = _NC * _NS

_IDX_W = 128
_K = 4
_CHUNK = _K * _IDX_W


def _gather_body(idx_hbm, table_hbm, out_hbm, idx_v, rows_a, rows_b,
                 gsem, oa_sem, ob_sem):
    b_total = out_hbm.shape[0]
    b_per_w = b_total // _NW
    n_chunks = b_per_w // _CHUNK
    n_pairs = n_chunks // 2
    idx_rows = b_per_w // _IDX_W

    wid = lax.axis_index("s") * _NC + lax.axis_index("c")
    row0 = pl.multiple_of(wid * b_per_w, _CHUNK)
    irow0 = pl.multiple_of(wid * idx_rows, 8)

    pltpu.sync_copy(idx_hbm.at[pl.ds(irow0, idx_rows)], idx_v)

    def out_slice(c):
        return out_hbm.at[pl.ds(pl.multiple_of(row0 + c * _CHUNK, _CHUNK),
                                _CHUNK)]

    def run_chunk(c, buf, osem):
        copies = []
        for j in range(_K):
            copies.append(
                pltpu.async_copy(
                    table_hbm.at[idx_v.at[c * _K + j]],
                    buf.at[pl.ds(j * _IDX_W, _IDX_W)],
                    gsem,
                )
            )
        for cp in copies:
            cp.wait()
        pltpu.async_copy(buf, out_slice(c), osem)

    def pair_body(i, carry):
        ca = 2 * i
        cb = 2 * i + 1

        @pl.when(i > 0)
        def _():
            pltpu.make_async_copy(rows_a, out_slice(ca), oa_sem).wait()

        run_chunk(ca, rows_a, oa_sem)

        @pl.when(i > 0)
        def _():
            pltpu.make_async_copy(rows_b, out_slice(cb), ob_sem).wait()

        run_chunk(cb, rows_b, ob_sem)
        return carry

    lax.fori_loop(0, n_pairs, pair_body, 0)

    last = n_chunks - 1
    pltpu.make_async_copy(rows_a, out_slice(last), oa_sem).wait()
    pltpu.make_async_copy(rows_b, out_slice(last), ob_sem).wait()


_SEQ_TC = 48


@jax.jit
def _impl(indices, table):
    batch, seq = indices.shape
    vocab, d = table.shape
    idx_t = indices.T
    seq_sc = seq - _SEQ_TC
    n_flat = batch * seq_sc

    tail = jnp.take(table, indices[:, seq_sc:], axis=0)

    idx2d = idx_t[:seq_sc].reshape(-1, _IDX_W)
    b_per_w = n_flat // _NW
    mesh = plsc.VectorSubcoreMesh(core_axis_name="c", subcore_axis_name="s")
    flat = pl.kernel(
        _gather_body,
        out_type=jax.ShapeDtypeStruct((n_flat, d), jnp.float32),
        mesh=mesh,
        scratch_types=[
            pltpu.VMEM((b_per_w // _IDX_W, _IDX_W), jnp.int32),
            pltpu.VMEM((_CHUNK, d), jnp.float32),
            pltpu.VMEM((_CHUNK, d), jnp.float32),
            pltpu.SemaphoreType.DMA,
            pltpu.SemaphoreType.DMA,
            pltpu.SemaphoreType.DMA,
        ],
        compiler_params=pltpu.CompilerParams(use_tc_tiling_on_sc=False),
    )(idx2d, table)

    head = flat.reshape(seq_sc, batch, d).transpose(1, 0, 2)
    return jnp.concatenate([head, tail], axis=1)


def kernel(indices, table):
    return _impl(indices, table)


# speedup vs baseline: 1.0788x; 1.0788x over previous
import jax
import jax.numpy as jnp
from jax import lax
from jax.experimental import pallas as pl
from jax.experimental.pallas import tpu as pltpu
from jax.experimental.pallas import tpu_sc as plsc

_NC = 2
_NS = 16
_NW = _NC * _NS

_IDX_W = 128
_K = 4
_CHUNK = _K * _IDX_W


def _gather_body(idx_hbm, table_hbm, out_hbm, idx_v, rows_a, rows_b,
                 gsem, oa_sem, ob_sem):
    b_total = out_hbm.shape[0]
    b_per_w = b_total // _NW
    n_chunks = b_per_w // _CHUNK
    n_pairs = n_chunks // 2
    idx_rows = b_per_w // _IDX_W

    wid = lax.axis_index("s") * _NC + lax.axis_index("c")
    row0 = pl.multiple_of(wid * b_per_w, _CHUNK)
    irow0 = pl.multiple_of(wid * idx_rows, 8)

    pltpu.sync_copy(idx_hbm.at[pl.ds(irow0, idx_rows)], idx_v)

    def out_slice(c):
        return out_hbm.at[pl.ds(pl.multiple_of(row0 + c * _CHUNK, _CHUNK),
                                _CHUNK)]

    def run_chunk(c, buf, osem):
        copies = []
        for j in range(_K):
            copies.append(
                pltpu.async_copy(
                    table_hbm.at[idx_v.at[c * _K + j]],
                    buf.at[pl.ds(j * _IDX_W, _IDX_W)],
                    gsem,
                )
            )
        for cp in copies:
            cp.wait()
        pltpu.async_copy(buf, out_slice(c), osem)

    def pair_body(i, carry):
        ca = 2 * i
        cb = 2 * i + 1

        @pl.when(i > 0)
        def _():
            pltpu.make_async_copy(rows_a, out_slice(ca), oa_sem).wait()

        run_chunk(ca, rows_a, oa_sem)

        @pl.when(i > 0)
        def _():
            pltpu.make_async_copy(rows_b, out_slice(cb), ob_sem).wait()

        run_chunk(cb, rows_b, ob_sem)
        return carry

    lax.fori_loop(0, n_pairs, pair_body, 0)

    last = n_chunks - 1
    pltpu.make_async_copy(rows_a, out_slice(last), oa_sem).wait()
    pltpu.make_async_copy(rows_b, out_slice(last), ob_sem).wait()


@jax.jit
def _impl(indices, table):
    batch, seq = indices.shape
    vocab, d = table.shape
    n_flat = batch * seq

    idx2d = indices.T.reshape(-1, _IDX_W)
    b_per_w = n_flat // _NW
    mesh = plsc.VectorSubcoreMesh(core_axis_name="c", subcore_axis_name="s")
    flat = pl.kernel(
        _gather_body,
        out_type=jax.ShapeDtypeStruct((n_flat, d), jnp.float32),
        mesh=mesh,
        scratch_types=[
            pltpu.VMEM((b_per_w // _IDX_W, _IDX_W), jnp.int32),
            pltpu.VMEM((_CHUNK, d), jnp.float32),
            pltpu.VMEM((_CHUNK, d), jnp.float32),
            pltpu.SemaphoreType.DMA,
            pltpu.SemaphoreType.DMA,
            pltpu.SemaphoreType.DMA,
        ],
        compiler_params=pltpu.CompilerParams(use_tc_tiling_on_sc=False),
    )(idx2d, table)

    return flat.reshape(seq, batch, d).transpose(1, 0, 2)


def kernel(indices, table):
    return _impl(indices, table)
